# Initial kernel scaffold; baseline (speedup 1.0000x reference)
import functools

import jax
import jax.numpy as jnp
from jax import lax
from jax.experimental import pallas as pl
from jax.experimental.pallas import tpu as pltpu
from jax.experimental.pallas import tpu_sc as plsc

N_NODES_ = 10000
N_EDGES_ = 160000
D_ = 256
DH_ = 128
N_TILES_ = 16
E_PER_TILE_ = N_EDGES_ // N_TILES_
CHUNK_ = 80
N_CHUNKS_ = E_PER_TILE_ // CHUNK_
ROWS_PER_TILE_ = N_NODES_ // N_TILES_


def _seg_sum_sc(hsplit, src_r, dst_r, zrows):
    mesh = plsc.VectorSubcoreMesh(core_axis_name="c", subcore_axis_name="s")

    @functools.partial(
        pl.kernel,
        mesh=mesh,
        out_type=jax.ShapeDtypeStruct((2, N_NODES_, DH_), jnp.float32),
        scratch_types=[
            pltpu.VMEM((125, CHUNK_), jnp.int32),
            pltpu.VMEM((125, CHUNK_), jnp.int32),
            pltpu.VMEM((CHUNK_, DH_), jnp.float32),
            pltpu.VMEM((ROWS_PER_TILE_, DH_), jnp.float32),
            pltpu.VMEM_SHARED((N_NODES_, DH_), jnp.float32),
            pltpu.SemaphoreType.DMA,
        ],
    )
    def seg_kernel(h_hbm, src_hbm, dst_hbm, z_hbm, out_hbm,
                   src_v, dst_v, rows_v, stage_v, acc, sem):
        c = lax.axis_index("c")
        s = lax.axis_index("s")

        pltpu.sync_copy(z_hbm, stage_v)
        pltpu.sync_copy(stage_v, acc.at[pl.ds(s * ROWS_PER_TILE_, ROWS_PER_TILE_)])
        pltpu.sync_copy(src_hbm.at[s], src_v)
        pltpu.sync_copy(dst_hbm.at[s], dst_v)
        plsc.subcore_barrier()

        def body(j, carry):
            pltpu.async_copy(h_hbm.at[c, src_v.at[j]], rows_v, sem).wait()
            pltpu.sync_copy(rows_v, acc.at[dst_v.at[j]], add=True)
            return carry

        lax.fori_loop(0, N_CHUNKS_, body, 0)
        plsc.subcore_barrier()

        pltpu.sync_copy(acc.at[pl.ds(s * ROWS_PER_TILE_, ROWS_PER_TILE_)], stage_v)
        pltpu.sync_copy(stage_v,
                        out_hbm.at[c, pl.ds(s * ROWS_PER_TILE_, ROWS_PER_TILE_)])

    return seg_kernel(hsplit, src_r, dst_r, zrows)


def _linear_tc(h2, W2, b2):
    BM = 2000

    def mm_kernel(h_ref, w_ref, b_ref, o_ref):
        dn = (((1,), (1,)), ((), ()))
        acc = lax.dot_general(h_ref[0], w_ref[0], dn,
                              preferred_element_type=jnp.float32)
        acc += lax.dot_general(h_ref[1], w_ref[1], dn,
                               preferred_element_type=jnp.float32)
        o_ref[...] = acc + b_ref[...]

    return pl.pallas_call(
        mm_kernel,
        grid=(N_NODES_ // BM,),
        in_specs=[
            pl.BlockSpec((2, BM, DH_), lambda i: (0, i, 0)),
            pl.BlockSpec((2, D_, DH_), lambda i: (0, 0, 0)),
            pl.BlockSpec((1, D_), lambda i: (0, 0)),
        ],
        out_specs=pl.BlockSpec((BM, D_), lambda i: (i, 0)),
        out_shape=jax.ShapeDtypeStruct((N_NODES_, D_), jnp.float32),
    )(h2, W2, b2)


def kernel(hidden, edge_index, W, b):
    src = edge_index[0].astype(jnp.int32).reshape(N_TILES_, N_CHUNKS_, CHUNK_)
    dst = edge_index[1].astype(jnp.int32).reshape(N_TILES_, N_CHUNKS_, CHUNK_)
    hsplit = jnp.stack([hidden[:, :DH_], hidden[:, DH_:]])
    zrows = jnp.zeros((ROWS_PER_TILE_, DH_), jnp.float32)
    h2 = _seg_sum_sc(hsplit, src, dst, zrows)
    W2 = jnp.stack([W[:, :DH_], W[:, DH_:]])
    return _linear_tc(h2, W2, b.reshape(1, D_))


# speedup vs baseline: 5.1154x; 5.1154x over previous
import functools

import jax
import jax.numpy as jnp
from jax import lax
from jax.experimental import pallas as pl
from jax.experimental.pallas import tpu as pltpu
from jax.experimental.pallas import tpu_sc as plsc

N_NODES_ = 10000
N_EDGES_ = 160000
D_ = 256
DH_ = 128
N_TILES_ = 16
E_PER_TILE_ = N_EDGES_ // N_TILES_
CHUNK_ = 80
N_CHUNKS_ = E_PER_TILE_ // CHUNK_
ROWS_A_ = 624
ROWS_B_ = 640


def _seg_sum_sc(hsplit, src_r, dst_r, zrows):
    mesh = plsc.VectorSubcoreMesh(core_axis_name="c", subcore_axis_name="s")

    @functools.partial(
        pl.kernel,
        mesh=mesh,
        out_type=jax.ShapeDtypeStruct((2, N_NODES_, DH_), jnp.float32),
        scratch_types=[
            pltpu.VMEM((125, CHUNK_), jnp.int32),
            pltpu.VMEM((125, CHUNK_), jnp.int32),
            pltpu.VMEM((CHUNK_, DH_), jnp.float32),
            pltpu.VMEM_SHARED((N_NODES_, DH_), jnp.float32),
            pltpu.SemaphoreType.DMA,
        ],
    )
    def seg_kernel(h_hbm, src_hbm, dst_hbm, z_hbm, out_hbm,
                   src_v, dst_v, rows_v, acc, sem):
        c = lax.axis_index("c")
        s = lax.axis_index("s")

        @pl.when(s < N_TILES_ - 1)
        def _():
            pltpu.sync_copy(z_hbm.at[pl.ds(0, ROWS_A_)],
                            acc.at[pl.ds(s * ROWS_A_, ROWS_A_)])

        @pl.when(s == N_TILES_ - 1)
        def _():
            pltpu.sync_copy(z_hbm, acc.at[pl.ds(15 * ROWS_A_, ROWS_B_)])
        pltpu.sync_copy(src_hbm.at[s], src_v)
        pltpu.sync_copy(dst_hbm.at[s], dst_v)
        plsc.subcore_barrier()

        def body(j, carry):
            pltpu.async_copy(h_hbm.at[c].at[src_v.at[j]], rows_v, sem).wait()
            pltpu.sync_copy(rows_v, acc.at[dst_v.at[j]], add=True)
            return carry

        lax.fori_loop(0, N_CHUNKS_, body, 0)
        plsc.subcore_barrier()

        @pl.when(s < N_TILES_ - 1)
        def _():
            pltpu.sync_copy(acc.at[pl.ds(s * ROWS_A_, ROWS_A_)],
                            out_hbm.at[c, pl.ds(s * ROWS_A_, ROWS_A_)])

        @pl.when(s == N_TILES_ - 1)
        def _():
            pltpu.sync_copy(acc.at[pl.ds(15 * ROWS_A_, ROWS_B_)],
                            out_hbm.at[c, pl.ds(15 * ROWS_A_, ROWS_B_)])

    return seg_kernel(hsplit, src_r, dst_r, zrows)


def _linear_tc(h2, W2, b2):
    BM = 2000

    def mm_kernel(h_ref, w_ref, b_ref, o_ref):
        dn = (((1,), (1,)), ((), ()))
        acc = lax.dot_general(h_ref[0], w_ref[0], dn,
                              preferred_element_type=jnp.float32)
        acc += lax.dot_general(h_ref[1], w_ref[1], dn,
                               preferred_element_type=jnp.float32)
        o_ref[...] = acc + b_ref[...]

    return pl.pallas_call(
        mm_kernel,
        grid=(N_NODES_ // BM,),
        in_specs=[
            pl.BlockSpec((2, BM, DH_), lambda i: (0, i, 0)),
            pl.BlockSpec((2, D_, DH_), lambda i: (0, 0, 0)),
            pl.BlockSpec((1, D_), lambda i: (0, 0)),
        ],
        out_specs=pl.BlockSpec((BM, D_), lambda i: (i, 0)),
        out_shape=jax.ShapeDtypeStruct((N_NODES_, D_), jnp.float32),
    )(h2, W2, b2)


def kernel(hidden, edge_index, W, b):
    src = edge_index[0].astype(jnp.int32).reshape(N_TILES_, N_CHUNKS_, CHUNK_)
    dst = edge_index[1].astype(jnp.int32).reshape(N_TILES_, N_CHUNKS_, CHUNK_)
    hsplit = jnp.stack([hidden[:, :DH_], hidden[:, DH_:]])
    zrows = jnp.zeros((ROWS_B_, DH_), jnp.float32)
    h2 = _seg_sum_sc(hsplit, src, dst, zrows)
    W2 = jnp.stack([W[:, :DH_], W[:, DH_:]])
    return _linear_tc(h2, W2, b.reshape(1, D_))


# speedup vs baseline: 6.6478x; 1.2996x over previous
import functools

import jax
import jax.numpy as jnp
from jax import lax
from jax.experimental import pallas as pl
from jax.experimental.pallas import tpu as pltpu
from jax.experimental.pallas import tpu_sc as plsc

N_NODES_ = 10000
N_EDGES_ = 160000
D_ = 256
DH_ = 128
N_TILES_ = 16
E_PER_TILE_ = N_EDGES_ // N_TILES_
CHUNK_ = 100
N_CHUNKS_ = E_PER_TILE_ // CHUNK_
OUTER_ = 5
IN_CH_ = N_CHUNKS_ // OUTER_
ROWS_A_ = 624
ROWS_B_ = 640


def _seg_sum_sc(hsplit, src_r, dst_r, zrows):
    mesh = plsc.VectorSubcoreMesh(core_axis_name="c", subcore_axis_name="s")

    @functools.partial(
        pl.kernel,
        mesh=mesh,
        out_type=jax.ShapeDtypeStruct((2, N_NODES_, DH_), jnp.float32),
        scratch_types=[
            pltpu.VMEM((IN_CH_, CHUNK_), jnp.int32),
            pltpu.VMEM((IN_CH_, CHUNK_), jnp.int32),
            pltpu.VMEM((CHUNK_, DH_), jnp.float32),
            pltpu.VMEM((CHUNK_, DH_), jnp.float32),
            pltpu.VMEM_SHARED((N_NODES_, DH_), jnp.float32),
            pltpu.SemaphoreType.DMA,
            pltpu.SemaphoreType.DMA,
        ],
    )
    def seg_kernel(h_hbm, src_hbm, dst_hbm, z_hbm, out_hbm,
                   src_v, dst_v, rows0_v, rows1_v, acc, sem0, sem1):
        c = lax.axis_index("c")
        s = lax.axis_index("s")

        @pl.when(s < N_TILES_ - 1)
        def _():
            pltpu.sync_copy(z_hbm.at[pl.ds(0, ROWS_A_)],
                            acc.at[pl.ds(s * ROWS_A_, ROWS_A_)])

        @pl.when(s == N_TILES_ - 1)
        def _():
            pltpu.sync_copy(z_hbm, acc.at[pl.ds(15 * ROWS_A_, ROWS_B_)])
        plsc.subcore_barrier()

        def gather(j, buf, sem):
            pltpu.async_copy(h_hbm.at[c].at[src_v.at[j]], buf, sem)

        def gwait(buf, sem):
            pltpu.make_async_copy(h_hbm.at[c].at[src_v.at[0]], buf, sem).wait()

        def scatter(j, buf):
            pltpu.sync_copy(buf, acc.at[dst_v.at[j]], add=True)

        def outer(o, carry):
            pltpu.sync_copy(src_hbm.at[s, o], src_v)
            pltpu.sync_copy(dst_hbm.at[s, o], dst_v)
            gather(0, rows0_v, sem0)

            def body(k, carry):
                j0 = 2 * k
                gwait(rows0_v, sem0)
                gather(j0 + 1, rows1_v, sem1)
                scatter(j0, rows0_v)
                gwait(rows1_v, sem1)

                @pl.when(k < IN_CH_ // 2 - 1)
                def _():
                    gather(j0 + 2, rows0_v, sem0)

                scatter(j0 + 1, rows1_v)
                return carry

            lax.fori_loop(0, IN_CH_ // 2, body, 0)
            return carry

        lax.fori_loop(0, OUTER_, outer, 0)
        plsc.subcore_barrier()

        @pl.when(s < N_TILES_ - 1)
        def _():
            pltpu.sync_copy(acc.at[pl.ds(s * ROWS_A_, ROWS_A_)],
                            out_hbm.at[c, pl.ds(s * ROWS_A_, ROWS_A_)])

        @pl.when(s == N_TILES_ - 1)
        def _():
            pltpu.sync_copy(acc.at[pl.ds(15 * ROWS_A_, ROWS_B_)],
                            out_hbm.at[c, pl.ds(15 * ROWS_A_, ROWS_B_)])

    return seg_kernel(hsplit, src_r, dst_r, zrows)


def _linear_tc(h2, W2, b2):
    BM = 2000

    def mm_kernel(h_ref, w_ref, b_ref, o_ref):
        dn = (((1,), (1,)), ((), ()))
        acc = lax.dot_general(h_ref[0], w_ref[0], dn,
                              preferred_element_type=jnp.float32)
        acc += lax.dot_general(h_ref[1], w_ref[1], dn,
                               preferred_element_type=jnp.float32)
        o_ref[...] = acc + b_ref[...]

    return pl.pallas_call(
        mm_kernel,
        grid=(N_NODES_ // BM,),
        in_specs=[
            pl.BlockSpec((2, BM, DH_), lambda i: (0, i, 0)),
            pl.BlockSpec((2, D_, DH_), lambda i: (0, 0, 0)),
            pl.BlockSpec((1, D_), lambda i: (0, 0)),
        ],
        out_specs=pl.BlockSpec((BM, D_), lambda i: (i, 0)),
        out_shape=jax.ShapeDtypeStruct((N_NODES_, D_), jnp.float32),
    )(h2, W2, b2)


def kernel(hidden, edge_index, W, b):
    src = edge_index[0].astype(jnp.int32).reshape(N_TILES_, OUTER_, IN_CH_, CHUNK_)
    dst = edge_index[1].astype(jnp.int32).reshape(N_TILES_, OUTER_, IN_CH_, CHUNK_)
    hsplit = jnp.stack([hidden[:, :DH_], hidden[:, DH_:]])
    zrows = jnp.zeros((ROWS_B_, DH_), jnp.float32)
    h2 = _seg_sum_sc(hsplit, src, dst, zrows)
    W2 = jnp.stack([W[:, :DH_], W[:, DH_:]])
    return _linear_tc(h2, W2, b.reshape(1, D_))
